# initial kernel scaffold (unmeasured)
import jax
import jax.numpy as jnp
from jax import lax
from jax.experimental import pallas as pl
from jax.experimental.pallas import tpu as pltpu

N_DEV = 4
B, S, D = 2, 512, 2048
DC = 512
DCS = DC // N_DEV
H, DH, DR = 16, 128, 32
BS = B * S
BF = jnp.bfloat16
F32 = jnp.float32


def _comm_body(x_ref, wdkv_ref, wuk_ref, wuv_ref, wqr_ref, wkr_ref,
               xb_ref, c_ref, wukf_ref, wuvf_ref, qr_ref, kr_ref,
               cbuf, kbuf, vbuf, csend, crecv, ksend, krecv, vsend, vrecv):
    my = lax.axis_index("i")
    right = lax.rem(my + 1, N_DEV)
    left = lax.rem(my + N_DEV - 1, N_DEV)

    barrier = pltpu.get_barrier_semaphore()
    for nbr in (left, right):
        pl.semaphore_signal(barrier, inc=1, device_id=(nbr,),
                            device_id_type=pl.DeviceIdType.MESH)
    pl.semaphore_wait(barrier, 2)

    xb = x_ref[...].reshape(BS, D).astype(BF)
    xb_ref[...] = xb
    cbuf[0] = jnp.dot(xb, wdkv_ref[...].astype(BF),
                      preferred_element_type=F32).astype(BF)
    kbuf[0] = wuk_ref[...].astype(BF)
    vbuf[0] = wuv_ref[...].astype(BF)

    for h in range(N_DEV - 1):
        rdmas = []
        for buf, ss, rr in ((cbuf, csend, crecv),
                            (kbuf, ksend, krecv),
                            (vbuf, vsend, vrecv)):
            r = pltpu.make_async_remote_copy(
                src_ref=buf.at[h], dst_ref=buf.at[h + 1],
                send_sem=ss.at[h], recv_sem=rr.at[h],
                device_id=(right,), device_id_type=pl.DeviceIdType.MESH)
            r.start()
            rdmas.append(r)
        if h == 0:
            qr_ref[...] = jnp.dot(xb, wqr_ref[...].astype(BF),
                                  preferred_element_type=F32).astype(BF)
            kr_ref[...] = jnp.dot(xb, wkr_ref[...].astype(BF),
                                  preferred_element_type=F32).astype(BF)
        for r in rdmas:
            r.wait()

    for j in range(N_DEV):
        origin = lax.rem(my - j + N_DEV, N_DEV)
        off = origin * DCS
        c_ref[:, pl.ds(off, DCS)] = cbuf[j]
        wukf_ref[pl.ds(off, DCS), :] = kbuf[j]
        wuvf_ref[pl.ds(off, DCS), :] = vbuf[j]


def _attn_body(xb_ref, c_ref, wuk_ref, wuv_ref, wq_ref, qr_ref, kr_ref,
               wo_ref, out_ref):
    h = pl.program_id(0)
    scale = (DH + DR) ** -0.5

    q = jnp.dot(xb_ref[...], wq_ref[...].astype(BF),
                preferred_element_type=F32)
    k = jnp.dot(c_ref[...], wuk_ref[...], preferred_element_type=F32)
    v = jnp.dot(c_ref[...], wuv_ref[...], preferred_element_type=F32)
    q3 = q.astype(BF).reshape(B, S, DH)
    k3 = k.astype(BF).reshape(B, S, DH)
    qr3 = qr_ref[...].reshape(B, S, DR)
    kr3 = kr_ref[...].reshape(B, S, DR)

    dn = (((2,), (2,)), ((0,), (0,)))
    scores = (lax.dot_general(q3, k3, dn, preferred_element_type=F32)
              + lax.dot_general(qr3, kr3, dn, preferred_element_type=F32))
    scores = scores * scale
    m = jnp.max(scores, axis=-1, keepdims=True)
    p = jnp.exp(scores - m)
    denom = jnp.sum(p, axis=-1, keepdims=True)

    dn_pv = (((2,), (1,)), ((0,), (0,)))
    o = lax.dot_general(p.astype(BF), v.astype(BF).reshape(B, S, DH), dn_pv,
                        preferred_element_type=F32)
    o = o / denom
    part = jnp.dot(o.reshape(BS, DH).astype(BF), wo_ref[...].astype(BF),
                   preferred_element_type=F32).reshape(B, S, D)

    @pl.when(h == 0)
    def _():
        out_ref[...] = part

    @pl.when(h != 0)
    def _():
        out_ref[...] += part


def kernel(x, Wdkv, Wuk, Wuv, Wq, Wqr, Wkr, Wo):
    xb, c, wukf, wuvf, qr, kr = pl.pallas_call(
        _comm_body,
        out_shape=(
            jax.ShapeDtypeStruct((BS, D), BF),
            jax.ShapeDtypeStruct((BS, DC), BF),
            jax.ShapeDtypeStruct((DC, D), BF),
            jax.ShapeDtypeStruct((DC, D), BF),
            jax.ShapeDtypeStruct((BS, H * DR), BF),
            jax.ShapeDtypeStruct((BS, DR), BF),
        ),
        in_specs=[pl.BlockSpec(memory_space=pltpu.VMEM)] * 6,
        out_specs=tuple([pl.BlockSpec(memory_space=pltpu.VMEM)] * 6),
        scratch_shapes=[
            pltpu.VMEM((N_DEV, BS, DCS), BF),
            pltpu.VMEM((N_DEV, DCS, D), BF),
            pltpu.VMEM((N_DEV, DCS, D), BF),
            pltpu.SemaphoreType.DMA((N_DEV - 1,)),
            pltpu.SemaphoreType.DMA((N_DEV - 1,)),
            pltpu.SemaphoreType.DMA((N_DEV - 1,)),
            pltpu.SemaphoreType.DMA((N_DEV - 1,)),
            pltpu.SemaphoreType.DMA((N_DEV - 1,)),
            pltpu.SemaphoreType.DMA((N_DEV - 1,)),
        ],
        compiler_params=pltpu.CompilerParams(collective_id=0),
    )(x, Wdkv, Wuk, Wuv, Wqr, Wkr)

    out = pl.pallas_call(
        _attn_body,
        grid=(H,),
        in_specs=[
            pl.BlockSpec((BS, D), lambda h: (0, 0)),
            pl.BlockSpec((BS, DC), lambda h: (0, 0)),
            pl.BlockSpec((DC, DH), lambda h: (0, h)),
            pl.BlockSpec((DC, DH), lambda h: (0, h)),
            pl.BlockSpec((D, DH), lambda h: (0, h)),
            pl.BlockSpec((BS, DR), lambda h: (0, h)),
            pl.BlockSpec((BS, DR), lambda h: (0, 0)),
            pl.BlockSpec((DH, D), lambda h: (h, 0)),
        ],
        out_specs=pl.BlockSpec((B, S, D), lambda h: (0, 0, 0)),
        out_shape=jax.ShapeDtypeStruct((B, S, D), F32),
    )(xb, c, wukf, wuvf, Wq, qr, kr, Wo)
    return out


# baseline (device time: 154961 ns/iter reference)
import jax
import jax.numpy as jnp
from jax import lax
from jax.experimental import pallas as pl
from jax.experimental.pallas import tpu as pltpu

N_DEV = 4
B, S, D = 2, 512, 2048
DC = 512
DCS = DC // N_DEV
H, DH, DR = 16, 128, 32
BS = B * S
BF = jnp.bfloat16
F32 = jnp.float32


def _comm_body(x_ref, wdkv_ref, wuk_ref, wuv_ref, wqr_ref, wkr_ref,
               xb_ref, c_ref, wukf_ref, wuvf_ref, qr_ref, kr_ref,
               cbuf, kbuf, vbuf, csend, crecv, ksend, krecv, vsend, vrecv):
    my = lax.axis_index("i")
    right = lax.rem(my + 1, N_DEV)
    left = lax.rem(my + N_DEV - 1, N_DEV)

    barrier = pltpu.get_barrier_semaphore()
    for nbr in (left, right):
        pl.semaphore_signal(barrier, inc=1, device_id=(nbr,),
                            device_id_type=pl.DeviceIdType.MESH)
    pl.semaphore_wait(barrier, 2)

    xb = x_ref[...].reshape(BS, D).astype(BF)
    xb_ref[...] = xb
    cbuf[0] = jnp.dot(xb, wdkv_ref[...].astype(BF),
                      preferred_element_type=F32).astype(BF)
    kbuf[0] = wuk_ref[...].astype(BF)
    vbuf[0] = wuv_ref[...].astype(BF)

    for h in range(N_DEV - 1):
        rdmas = []
        for buf, ss, rr in ((cbuf, csend, crecv),
                            (kbuf, ksend, krecv),
                            (vbuf, vsend, vrecv)):
            r = pltpu.make_async_remote_copy(
                src_ref=buf.at[h], dst_ref=buf.at[h + 1],
                send_sem=ss.at[h], recv_sem=rr.at[h],
                device_id=(right,), device_id_type=pl.DeviceIdType.MESH)
            r.start()
            rdmas.append(r)
        if h == 0:
            qr_full = jnp.dot(xb, wqr_ref[...].astype(BF),
                              preferred_element_type=F32).astype(BF)
            for hd in range(H):
                qr_ref[hd] = qr_full[:, hd * DR:(hd + 1) * DR]
            kr_ref[...] = jnp.dot(xb, wkr_ref[...].astype(BF),
                                  preferred_element_type=F32).astype(BF)
        for r in rdmas:
            r.wait()

    for j in range(N_DEV):
        origin = lax.rem(my - j + N_DEV, N_DEV)
        off = origin * DCS
        c_ref[:, pl.ds(off, DCS)] = cbuf[j]
        wukf_ref[pl.ds(off, DCS), :] = kbuf[j]
        wuvf_ref[pl.ds(off, DCS), :] = vbuf[j]


def _attn_body(xb_ref, c_ref, wuk_ref, wuv_ref, wq_ref, qr_ref, kr_ref,
               wo_ref, out_ref):
    h = pl.program_id(0)
    scale = (DH + DR) ** -0.5

    q = jnp.dot(xb_ref[...], wq_ref[...].astype(BF),
                preferred_element_type=F32)
    k = jnp.dot(c_ref[...], wuk_ref[...], preferred_element_type=F32)
    v = jnp.dot(c_ref[...], wuv_ref[...], preferred_element_type=F32)
    q3 = q.astype(BF).reshape(B, S, DH)
    k3 = k.astype(BF).reshape(B, S, DH)
    qr3 = qr_ref[0].reshape(B, S, DR)
    kr3 = kr_ref[...].reshape(B, S, DR)

    dn = (((2,), (2,)), ((0,), (0,)))
    scores = (lax.dot_general(q3, k3, dn, preferred_element_type=F32)
              + lax.dot_general(qr3, kr3, dn, preferred_element_type=F32))
    scores = scores * scale
    m = jnp.max(scores, axis=-1, keepdims=True)
    p = jnp.exp(scores - m)
    denom = jnp.sum(p, axis=-1, keepdims=True)

    dn_pv = (((2,), (1,)), ((0,), (0,)))
    o = lax.dot_general(p.astype(BF), v.astype(BF).reshape(B, S, DH), dn_pv,
                        preferred_element_type=F32)
    o = o / denom
    part = jnp.dot(o.reshape(BS, DH).astype(BF), wo_ref[...].astype(BF),
                   preferred_element_type=F32).reshape(B, S, D)

    @pl.when(h == 0)
    def _():
        out_ref[...] = part

    @pl.when(h != 0)
    def _():
        out_ref[...] += part


def kernel(x, Wdkv, Wuk, Wuv, Wq, Wqr, Wkr, Wo):
    xb, c, wukf, wuvf, qr, kr = pl.pallas_call(
        _comm_body,
        out_shape=(
            jax.ShapeDtypeStruct((BS, D), BF),
            jax.ShapeDtypeStruct((BS, DC), BF),
            jax.ShapeDtypeStruct((DC, D), BF),
            jax.ShapeDtypeStruct((DC, D), BF),
            jax.ShapeDtypeStruct((H, BS, DR), BF),
            jax.ShapeDtypeStruct((BS, DR), BF),
        ),
        in_specs=[pl.BlockSpec(memory_space=pltpu.VMEM)] * 6,
        out_specs=tuple([pl.BlockSpec(memory_space=pltpu.VMEM)] * 6),
        scratch_shapes=[
            pltpu.VMEM((N_DEV, BS, DCS), BF),
            pltpu.VMEM((N_DEV, DCS, D), BF),
            pltpu.VMEM((N_DEV, DCS, D), BF),
            pltpu.SemaphoreType.DMA((N_DEV - 1,)),
            pltpu.SemaphoreType.DMA((N_DEV - 1,)),
            pltpu.SemaphoreType.DMA((N_DEV - 1,)),
            pltpu.SemaphoreType.DMA((N_DEV - 1,)),
            pltpu.SemaphoreType.DMA((N_DEV - 1,)),
            pltpu.SemaphoreType.DMA((N_DEV - 1,)),
        ],
        compiler_params=pltpu.CompilerParams(collective_id=0),
    )(x, Wdkv, Wuk, Wuv, Wqr, Wkr)

    out = pl.pallas_call(
        _attn_body,
        grid=(H,),
        in_specs=[
            pl.BlockSpec((BS, D), lambda h: (0, 0)),
            pl.BlockSpec((BS, DC), lambda h: (0, 0)),
            pl.BlockSpec((DC, DH), lambda h: (0, h)),
            pl.BlockSpec((DC, DH), lambda h: (0, h)),
            pl.BlockSpec((D, DH), lambda h: (0, h)),
            pl.BlockSpec((1, BS, DR), lambda h: (h, 0, 0)),
            pl.BlockSpec((BS, DR), lambda h: (0, 0)),
            pl.BlockSpec((DH, D), lambda h: (h, 0)),
        ],
        out_specs=pl.BlockSpec((B, S, D), lambda h: (0, 0, 0)),
        out_shape=jax.ShapeDtypeStruct((B, S, D), F32),
    )(xb, c, wukf, wuvf, Wq, qr, kr, Wo)
    return out


# device time: 136489 ns/iter; 1.1353x vs baseline; 1.1353x over previous
import jax
import jax.numpy as jnp
from jax import lax
from jax.experimental import pallas as pl
from jax.experimental.pallas import tpu as pltpu

N_DEV = 4
B, S, D = 2, 512, 2048
DC = 512
DCS = DC // N_DEV
H, DH, DR = 16, 128, 32
BS = B * S
HPB = H // 4
BF = jnp.bfloat16
F32 = jnp.float32


def _comm_body(x_ref, wdkv_ref, wuk_ref, wuv_ref,
               k_ref, v_ref,
               cbuf, kbuf, vbuf, csend, crecv, ksend, krecv, vsend, vrecv):
    my = lax.axis_index("i")
    right = lax.rem(my + 1, N_DEV)
    left = lax.rem(my + N_DEV - 1, N_DEV)

    barrier = pltpu.get_barrier_semaphore()
    for nbr in (left, right):
        pl.semaphore_signal(barrier, inc=1, device_id=(nbr,),
                            device_id_type=pl.DeviceIdType.MESH)
    pl.semaphore_wait(barrier, 2)

    xb = x_ref[...].reshape(BS, D).astype(BF)
    cbuf[0] = jnp.dot(xb, wdkv_ref[...].astype(BF),
                      preferred_element_type=F32).astype(BF)
    kbuf[0] = wuk_ref[...].astype(BF)
    vbuf[0] = wuv_ref[...].astype(BF)

    k_acc = None
    v_acc = None
    for h in range(N_DEV - 1):
        rdmas = []
        for buf, ss, rr in ((cbuf, csend, crecv),
                            (kbuf, ksend, krecv),
                            (vbuf, vsend, vrecv)):
            r = pltpu.make_async_remote_copy(
                src_ref=buf.at[h], dst_ref=buf.at[h + 1],
                send_sem=ss.at[h], recv_sem=rr.at[h],
                device_id=(right,), device_id_type=pl.DeviceIdType.MESH)
            r.start()
            rdmas.append(r)
        if h == 0:
            k_acc = jnp.dot(cbuf[0], kbuf[0], preferred_element_type=F32)
            v_acc = jnp.dot(cbuf[0], vbuf[0], preferred_element_type=F32)
        else:
            k_acc += jnp.dot(cbuf[h], kbuf[h], preferred_element_type=F32)
            v_acc += jnp.dot(cbuf[h], vbuf[h], preferred_element_type=F32)
        for r in rdmas:
            r.wait()

    j = N_DEV - 1
    k_acc += jnp.dot(cbuf[j], kbuf[j], preferred_element_type=F32)
    v_acc += jnp.dot(cbuf[j], vbuf[j], preferred_element_type=F32)
    k_ref[...] = k_acc.astype(BF)
    v_ref[...] = v_acc.astype(BF)


def _proj_body(x_ref, wq_ref, wqr_ref, wkr_ref, q_ref, qr_ref, kr_ref):
    j = pl.program_id(0)
    xb = x_ref[...].reshape(BS, D).astype(BF)
    q_ref[...] = jnp.dot(xb, wq_ref[...].astype(BF),
                         preferred_element_type=F32).astype(BF)
    qr_blk = jnp.dot(xb, wqr_ref[...].astype(BF),
                     preferred_element_type=F32).astype(BF)
    for hd in range(HPB):
        qr_ref[hd] = qr_blk[:, hd * DR:(hd + 1) * DR]

    @pl.when(j == 0)
    def _():
        kr_ref[...] = jnp.dot(xb, wkr_ref[...].astype(BF),
                              preferred_element_type=F32).astype(BF)


def _attn_body(q_ref, k_ref, v_ref, qr_ref, kr_ref, o_ref):
    scale = (DH + DR) ** -0.5
    q3 = q_ref[...].reshape(B, S, DH)
    k3 = k_ref[...].reshape(B, S, DH)
    qr3 = qr_ref[0].reshape(B, S, DR)
    kr3 = kr_ref[...].reshape(B, S, DR)

    dn = (((2,), (2,)), ((0,), (0,)))
    scores = (lax.dot_general(q3, k3, dn, preferred_element_type=F32)
              + lax.dot_general(qr3, kr3, dn, preferred_element_type=F32))
    scores = scores * scale
    m = jnp.max(scores, axis=-1, keepdims=True)
    p = jnp.exp(scores - m)
    denom = jnp.sum(p, axis=-1, keepdims=True)

    dn_pv = (((2,), (1,)), ((0,), (0,)))
    o = lax.dot_general(p.astype(BF), v_ref[...].reshape(B, S, DH), dn_pv,
                        preferred_element_type=F32)
    o = o / denom
    o_ref[...] = o.reshape(BS, DH).astype(BF)


def _out_body(o_ref, wo_ref, out_ref):
    j = pl.program_id(0)
    part = jnp.dot(o_ref[...], wo_ref[...].astype(BF),
                   preferred_element_type=F32).reshape(B, S, D)

    @pl.when(j == 0)
    def _():
        out_ref[...] = part

    @pl.when(j != 0)
    def _():
        out_ref[...] += part


def kernel(x, Wdkv, Wuk, Wuv, Wq, Wqr, Wkr, Wo):
    k, v = pl.pallas_call(
        _comm_body,
        out_shape=(
            jax.ShapeDtypeStruct((BS, D), BF),
            jax.ShapeDtypeStruct((BS, D), BF),
        ),
        in_specs=[pl.BlockSpec(memory_space=pltpu.VMEM)] * 4,
        out_specs=tuple([pl.BlockSpec(memory_space=pltpu.VMEM)] * 2),
        scratch_shapes=[
            pltpu.VMEM((N_DEV, BS, DCS), BF),
            pltpu.VMEM((N_DEV, DCS, D), BF),
            pltpu.VMEM((N_DEV, DCS, D), BF),
            pltpu.SemaphoreType.DMA((N_DEV - 1,)),
            pltpu.SemaphoreType.DMA((N_DEV - 1,)),
            pltpu.SemaphoreType.DMA((N_DEV - 1,)),
            pltpu.SemaphoreType.DMA((N_DEV - 1,)),
            pltpu.SemaphoreType.DMA((N_DEV - 1,)),
            pltpu.SemaphoreType.DMA((N_DEV - 1,)),
        ],
        compiler_params=pltpu.CompilerParams(collective_id=0),
    )(x, Wdkv, Wuk, Wuv)

    q, qr, kr = pl.pallas_call(
        _proj_body,
        grid=(4,),
        in_specs=[
            pl.BlockSpec((B, S, D), lambda j: (0, 0, 0)),
            pl.BlockSpec((D, D // 4), lambda j: (0, j)),
            pl.BlockSpec((D, H * DR // 4), lambda j: (0, j)),
            pl.BlockSpec((D, DR), lambda j: (0, 0)),
        ],
        out_specs=(
            pl.BlockSpec((BS, D // 4), lambda j: (0, j)),
            pl.BlockSpec((HPB, BS, DR), lambda j: (j, 0, 0)),
            pl.BlockSpec((BS, DR), lambda j: (0, 0)),
        ),
        out_shape=(
            jax.ShapeDtypeStruct((BS, D), BF),
            jax.ShapeDtypeStruct((H, BS, DR), BF),
            jax.ShapeDtypeStruct((BS, DR), BF),
        ),
    )(x, Wq, Wqr, Wkr)

    o = pl.pallas_call(
        _attn_body,
        grid=(H,),
        in_specs=[
            pl.BlockSpec((BS, DH), lambda h: (0, h)),
            pl.BlockSpec((BS, DH), lambda h: (0, h)),
            pl.BlockSpec((BS, DH), lambda h: (0, h)),
            pl.BlockSpec((1, BS, DR), lambda h: (h, 0, 0)),
            pl.BlockSpec((BS, DR), lambda h: (0, 0)),
        ],
        out_specs=pl.BlockSpec((BS, DH), lambda h: (0, h)),
        out_shape=jax.ShapeDtypeStruct((BS, D), BF),
    )(q, k, v, qr, kr)

    out = pl.pallas_call(
        _out_body,
        grid=(4,),
        in_specs=[
            pl.BlockSpec((BS, D // 4), lambda j: (0, j)),
            pl.BlockSpec((D // 4, D), lambda j: (j, 0)),
        ],
        out_specs=pl.BlockSpec((B, S, D), lambda j: (0, 0, 0)),
        out_shape=jax.ShapeDtypeStruct((B, S, D), F32),
    )(o, Wo)
    return out


# device time: 128067 ns/iter; 1.2100x vs baseline; 1.0658x over previous
import jax
import jax.numpy as jnp
from jax import lax
from jax.experimental import pallas as pl
from jax.experimental.pallas import tpu as pltpu

N_DEV = 4
B, S, D = 2, 512, 2048
DC = 512
DCS = DC // N_DEV
H, DH, DR = 16, 128, 32
BS = B * S
HPB = H // 4
BF = jnp.bfloat16
F32 = jnp.float32


def _comm_body(x_ref, wdkv_ref, wuk_ref, wuv_ref,
               xb_ref, k_ref, v_ref,
               cbuf, kbuf, vbuf, csend, crecv, ksend, krecv, vsend, vrecv):
    my = lax.axis_index("i")
    right = lax.rem(my + 1, N_DEV)
    left = lax.rem(my + N_DEV - 1, N_DEV)

    barrier = pltpu.get_barrier_semaphore()
    for nbr in (left, right):
        pl.semaphore_signal(barrier, inc=1, device_id=(nbr,),
                            device_id_type=pl.DeviceIdType.MESH)
    pl.semaphore_wait(barrier, 2)

    xb = x_ref[...].reshape(BS, D).astype(BF)
    xb_ref[...] = xb
    cbuf[0] = jnp.dot(xb, wdkv_ref[...].astype(BF),
                      preferred_element_type=F32).astype(BF)
    kbuf[0] = wuk_ref[...].astype(BF)
    vbuf[0] = wuv_ref[...].astype(BF)

    k_acc = None
    v_acc = None
    for h in range(N_DEV - 1):
        rdmas = []
        for buf, ss, rr in ((cbuf, csend, crecv),
                            (kbuf, ksend, krecv),
                            (vbuf, vsend, vrecv)):
            r = pltpu.make_async_remote_copy(
                src_ref=buf.at[h], dst_ref=buf.at[h + 1],
                send_sem=ss.at[h], recv_sem=rr.at[h],
                device_id=(right,), device_id_type=pl.DeviceIdType.MESH)
            r.start()
            rdmas.append(r)
        if h == 0:
            k_acc = jnp.dot(cbuf[0], kbuf[0], preferred_element_type=F32)
            v_acc = jnp.dot(cbuf[0], vbuf[0], preferred_element_type=F32)
        else:
            k_acc += jnp.dot(cbuf[h], kbuf[h], preferred_element_type=F32)
            v_acc += jnp.dot(cbuf[h], vbuf[h], preferred_element_type=F32)
        for r in rdmas:
            r.wait()

    j = N_DEV - 1
    k_acc += jnp.dot(cbuf[j], kbuf[j], preferred_element_type=F32)
    v_acc += jnp.dot(cbuf[j], vbuf[j], preferred_element_type=F32)
    k_ref[...] = k_acc.astype(BF)
    v_ref[...] = v_acc.astype(BF)


def _proj_body(xb_ref, wq_ref, wqr_ref, wkr_ref, q_ref, qr_ref, kr_ref):
    j = pl.program_id(0)
    scale = (DH + DR) ** -0.5
    xb = xb_ref[...]
    q_ref[...] = (jnp.dot(xb, wq_ref[...].astype(BF),
                          preferred_element_type=F32) * scale).astype(BF)
    qr_blk = (jnp.dot(xb, wqr_ref[...].astype(BF),
                      preferred_element_type=F32) * scale).astype(BF)
    for hd in range(HPB):
        qr_ref[hd] = qr_blk[:, hd * DR:(hd + 1) * DR]

    @pl.when(j == 0)
    def _():
        kr_ref[...] = jnp.dot(xb, wkr_ref[...].astype(BF),
                              preferred_element_type=F32).astype(BF)


def _attn_body(q_ref, k_ref, v_ref, qr_ref, kr_ref, o_ref):
    q3 = q_ref[...].reshape(B, S, DH)
    k3 = k_ref[...].reshape(B, S, DH)
    qr3 = qr_ref[0].reshape(B, S, DR)
    kr3 = kr_ref[...].reshape(B, S, DR)

    dn = (((2,), (2,)), ((0,), (0,)))
    scores = (lax.dot_general(q3, k3, dn, preferred_element_type=F32)
              + lax.dot_general(qr3, kr3, dn, preferred_element_type=F32))
    p = jnp.exp(scores)
    denom = jnp.sum(p, axis=-1, keepdims=True)

    dn_pv = (((2,), (1,)), ((0,), (0,)))
    o = lax.dot_general(p.astype(BF), v_ref[...].reshape(B, S, DH), dn_pv,
                        preferred_element_type=F32)
    o = o / denom
    o_ref[...] = o.reshape(BS, DH).astype(BF)


def _out_body(o_ref, wo_ref, out_ref):
    j = pl.program_id(0)
    part = jnp.dot(o_ref[...], wo_ref[...].astype(BF),
                   preferred_element_type=F32).reshape(B, S, D)

    @pl.when(j == 0)
    def _():
        out_ref[...] = part

    @pl.when(j != 0)
    def _():
        out_ref[...] += part


def kernel(x, Wdkv, Wuk, Wuv, Wq, Wqr, Wkr, Wo):
    xb, k, v = pl.pallas_call(
        _comm_body,
        out_shape=(
            jax.ShapeDtypeStruct((BS, D), BF),
            jax.ShapeDtypeStruct((BS, D), BF),
            jax.ShapeDtypeStruct((BS, D), BF),
        ),
        in_specs=[pl.BlockSpec(memory_space=pltpu.VMEM)] * 4,
        out_specs=tuple([pl.BlockSpec(memory_space=pltpu.VMEM)] * 3),
        scratch_shapes=[
            pltpu.VMEM((N_DEV, BS, DCS), BF),
            pltpu.VMEM((N_DEV, DCS, D), BF),
            pltpu.VMEM((N_DEV, DCS, D), BF),
            pltpu.SemaphoreType.DMA((N_DEV - 1,)),
            pltpu.SemaphoreType.DMA((N_DEV - 1,)),
            pltpu.SemaphoreType.DMA((N_DEV - 1,)),
            pltpu.SemaphoreType.DMA((N_DEV - 1,)),
            pltpu.SemaphoreType.DMA((N_DEV - 1,)),
            pltpu.SemaphoreType.DMA((N_DEV - 1,)),
        ],
        compiler_params=pltpu.CompilerParams(collective_id=0),
    )(x, Wdkv, Wuk, Wuv)

    q, qr, kr = pl.pallas_call(
        _proj_body,
        grid=(4,),
        in_specs=[
            pl.BlockSpec((BS, D), lambda j: (0, 0)),
            pl.BlockSpec((D, D // 4), lambda j: (0, j)),
            pl.BlockSpec((D, H * DR // 4), lambda j: (0, j)),
            pl.BlockSpec((D, DR), lambda j: (0, 0)),
        ],
        out_specs=(
            pl.BlockSpec((BS, D // 4), lambda j: (0, j)),
            pl.BlockSpec((HPB, BS, DR), lambda j: (j, 0, 0)),
            pl.BlockSpec((BS, DR), lambda j: (0, 0)),
        ),
        out_shape=(
            jax.ShapeDtypeStruct((BS, D), BF),
            jax.ShapeDtypeStruct((H, BS, DR), BF),
            jax.ShapeDtypeStruct((BS, DR), BF),
        ),
    )(xb, Wq, Wqr, Wkr)

    o = pl.pallas_call(
        _attn_body,
        grid=(H,),
        in_specs=[
            pl.BlockSpec((BS, DH), lambda h: (0, h)),
            pl.BlockSpec((BS, DH), lambda h: (0, h)),
            pl.BlockSpec((BS, DH), lambda h: (0, h)),
            pl.BlockSpec((1, BS, DR), lambda h: (h, 0, 0)),
            pl.BlockSpec((BS, DR), lambda h: (0, 0)),
        ],
        out_specs=pl.BlockSpec((BS, DH), lambda h: (0, h)),
        out_shape=jax.ShapeDtypeStruct((BS, D), BF),
    )(q, k, v, qr, kr)

    out = pl.pallas_call(
        _out_body,
        grid=(2,),
        in_specs=[
            pl.BlockSpec((BS, D // 2), lambda j: (0, j)),
            pl.BlockSpec((D // 2, D), lambda j: (j, 0)),
        ],
        out_specs=pl.BlockSpec((B, S, D), lambda j: (0, 0, 0)),
        out_shape=jax.ShapeDtypeStruct((B, S, D), F32),
    )(o, Wo)
    return out


# device time: 126550 ns/iter; 1.2245x vs baseline; 1.0120x over previous
import jax
import jax.numpy as jnp
from jax import lax
from jax.experimental import pallas as pl
from jax.experimental.pallas import tpu as pltpu

N_DEV = 4
B, S, D = 2, 512, 2048
DC = 512
DCS = DC // N_DEV
H, DH, DR = 16, 128, 32
BS = B * S
HPB = H // 4
BF = jnp.bfloat16
F32 = jnp.float32


def _cast_body(x_ref, xb_ref):
    xb_ref[...] = x_ref[...].reshape(BS, D).astype(BF)


def _comm_body(xb_ref, wdkv_ref, wuk_ref, wuv_ref,
               k_ref, v_ref,
               cbuf, kbuf, vbuf, csend, crecv, ksend, krecv, vsend, vrecv):
    my = lax.axis_index("i")
    right = lax.rem(my + 1, N_DEV)
    left = lax.rem(my + N_DEV - 1, N_DEV)

    barrier = pltpu.get_barrier_semaphore()
    for nbr in (left, right):
        pl.semaphore_signal(barrier, inc=1, device_id=(nbr,),
                            device_id_type=pl.DeviceIdType.MESH)
    pl.semaphore_wait(barrier, 2)

    cbuf[0] = jnp.dot(xb_ref[...], wdkv_ref[...].astype(BF),
                      preferred_element_type=F32).astype(BF)
    kbuf[0] = wuk_ref[...].astype(BF)
    vbuf[0] = wuv_ref[...].astype(BF)

    k_acc = None
    v_acc = None
    for h in range(N_DEV - 1):
        rdmas = []
        for buf, ss, rr in ((cbuf, csend, crecv),
                            (kbuf, ksend, krecv),
                            (vbuf, vsend, vrecv)):
            r = pltpu.make_async_remote_copy(
                src_ref=buf.at[h], dst_ref=buf.at[h + 1],
                send_sem=ss.at[h], recv_sem=rr.at[h],
                device_id=(right,), device_id_type=pl.DeviceIdType.MESH)
            r.start()
            rdmas.append(r)
        if h == 0:
            k_acc = jnp.dot(cbuf[0], kbuf[0], preferred_element_type=F32)
            v_acc = jnp.dot(cbuf[0], vbuf[0], preferred_element_type=F32)
        else:
            k_acc += jnp.dot(cbuf[h], kbuf[h], preferred_element_type=F32)
            v_acc += jnp.dot(cbuf[h], vbuf[h], preferred_element_type=F32)
        for r in rdmas:
            r.wait()

    j = N_DEV - 1
    k_acc += jnp.dot(cbuf[j], kbuf[j], preferred_element_type=F32)
    v_acc += jnp.dot(cbuf[j], vbuf[j], preferred_element_type=F32)
    k_ref[...] = k_acc.astype(BF)
    v_ref[...] = v_acc.astype(BF)


def _proj_body(xb_ref, wq_ref, wqr_ref, wkr_ref, q_ref, qr_ref, kr_ref):
    j = pl.program_id(0)
    scale = (DH + DR) ** -0.5
    xb = xb_ref[...]
    q_ref[...] = (jnp.dot(xb, wq_ref[...].astype(BF),
                          preferred_element_type=F32) * scale).astype(BF)
    qr_blk = (jnp.dot(xb, wqr_ref[...].astype(BF),
                      preferred_element_type=F32) * scale).astype(BF)
    for hd in range(HPB):
        qr_ref[hd] = qr_blk[:, hd * DR:(hd + 1) * DR]

    @pl.when(j == 0)
    def _():
        kr_ref[...] = jnp.dot(xb, wkr_ref[...].astype(BF),
                              preferred_element_type=F32).astype(BF)


def _attn_body(q_ref, k_ref, v_ref, qr_ref, kr_ref, o_ref):
    q3 = q_ref[...].reshape(B, S, DH)
    k3 = k_ref[...].reshape(B, S, DH)
    qr3 = qr_ref[0].reshape(B, S, DR)
    kr3 = kr_ref[...].reshape(B, S, DR)

    dn = (((2,), (2,)), ((0,), (0,)))
    scores = (lax.dot_general(q3, k3, dn, preferred_element_type=F32)
              + lax.dot_general(qr3, kr3, dn, preferred_element_type=F32))
    p = jnp.exp(scores)
    denom = jnp.sum(p, axis=-1, keepdims=True)

    dn_pv = (((2,), (1,)), ((0,), (0,)))
    o = lax.dot_general(p.astype(BF), v_ref[...].reshape(B, S, DH), dn_pv,
                        preferred_element_type=F32)
    o = o / denom
    o_ref[...] = o.reshape(BS, DH).astype(BF)


def _out_body(o_ref, wo_ref, out_ref):
    j = pl.program_id(0)
    part = jnp.dot(o_ref[...], wo_ref[...].astype(BF),
                   preferred_element_type=F32).reshape(B, S, D)

    @pl.when(j == 0)
    def _():
        out_ref[...] = part

    @pl.when(j != 0)
    def _():
        out_ref[...] += part


def kernel(x, Wdkv, Wuk, Wuv, Wq, Wqr, Wkr, Wo):
    xb = pl.pallas_call(
        _cast_body,
        out_shape=jax.ShapeDtypeStruct((BS, D), BF),
        in_specs=[pl.BlockSpec(memory_space=pltpu.VMEM)],
        out_specs=pl.BlockSpec(memory_space=pltpu.VMEM),
    )(x)

    k, v = pl.pallas_call(
        _comm_body,
        out_shape=(
            jax.ShapeDtypeStruct((BS, D), BF),
            jax.ShapeDtypeStruct((BS, D), BF),
        ),
        in_specs=[pl.BlockSpec(memory_space=pltpu.VMEM)] * 4,
        out_specs=tuple([pl.BlockSpec(memory_space=pltpu.VMEM)] * 2),
        scratch_shapes=[
            pltpu.VMEM((N_DEV, BS, DCS), BF),
            pltpu.VMEM((N_DEV, DCS, D), BF),
            pltpu.VMEM((N_DEV, DCS, D), BF),
            pltpu.SemaphoreType.DMA((N_DEV - 1,)),
            pltpu.SemaphoreType.DMA((N_DEV - 1,)),
            pltpu.SemaphoreType.DMA((N_DEV - 1,)),
            pltpu.SemaphoreType.DMA((N_DEV - 1,)),
            pltpu.SemaphoreType.DMA((N_DEV - 1,)),
            pltpu.SemaphoreType.DMA((N_DEV - 1,)),
        ],
        compiler_params=pltpu.CompilerParams(collective_id=0),
    )(xb, Wdkv, Wuk, Wuv)

    q, qr, kr = pl.pallas_call(
        _proj_body,
        grid=(4,),
        in_specs=[
            pl.BlockSpec((BS, D), lambda j: (0, 0)),
            pl.BlockSpec((D, D // 4), lambda j: (0, j)),
            pl.BlockSpec((D, H * DR // 4), lambda j: (0, j)),
            pl.BlockSpec((D, DR), lambda j: (0, 0)),
        ],
        out_specs=(
            pl.BlockSpec((BS, D // 4), lambda j: (0, j)),
            pl.BlockSpec((HPB, BS, DR), lambda j: (j, 0, 0)),
            pl.BlockSpec((BS, DR), lambda j: (0, 0)),
        ),
        out_shape=(
            jax.ShapeDtypeStruct((BS, D), BF),
            jax.ShapeDtypeStruct((H, BS, DR), BF),
            jax.ShapeDtypeStruct((BS, DR), BF),
        ),
    )(xb, Wq, Wqr, Wkr)

    o = pl.pallas_call(
        _attn_body,
        grid=(H,),
        in_specs=[
            pl.BlockSpec((BS, DH), lambda h: (0, h)),
            pl.BlockSpec((BS, DH), lambda h: (0, h)),
            pl.BlockSpec((BS, DH), lambda h: (0, h)),
            pl.BlockSpec((1, BS, DR), lambda h: (h, 0, 0)),
            pl.BlockSpec((BS, DR), lambda h: (0, 0)),
        ],
        out_specs=pl.BlockSpec((BS, DH), lambda h: (0, h)),
        out_shape=jax.ShapeDtypeStruct((BS, D), BF),
    )(q, k, v, qr, kr)

    out = pl.pallas_call(
        _out_body,
        grid=(2,),
        in_specs=[
            pl.BlockSpec((BS, D // 2), lambda j: (0, j)),
            pl.BlockSpec((D // 2, D), lambda j: (j, 0)),
        ],
        out_specs=pl.BlockSpec((B, S, D), lambda j: (0, 0, 0)),
        out_shape=jax.ShapeDtypeStruct((B, S, D), F32),
    )(o, Wo)
    return out


# device time: 107615 ns/iter; 1.4400x vs baseline; 1.1760x over previous
import jax
import jax.numpy as jnp
from jax import lax
from jax.experimental import pallas as pl
from jax.experimental.pallas import tpu as pltpu

N_DEV = 4
B, S, D = 2, 512, 2048
DC = 512
DCS = DC // N_DEV
H, DH, DR = 16, 128, 32
BS = B * S
HPB = H // 4
BF = jnp.bfloat16
F32 = jnp.float32


def _comm_body(x_ref, wdkv_ref, wuk_ref, wuv_ref,
               xb_ref, k_ref, v_ref,
               cbufr, cbufl, kbuf, vbuf,
               crs, crr, cls, clr, krs, krr, vls, vlr):
    my = lax.axis_index("i")
    right = lax.rem(my + 1, N_DEV)
    left = lax.rem(my + N_DEV - 1, N_DEV)

    barrier = pltpu.get_barrier_semaphore()
    for nbr in (left, right):
        pl.semaphore_signal(barrier, inc=1, device_id=(nbr,),
                            device_id_type=pl.DeviceIdType.MESH)
    pl.semaphore_wait(barrier, 2)

    xb = x_ref[...].reshape(BS, D).astype(BF)
    xb_ref[...] = xb
    c0 = jnp.dot(xb, wdkv_ref[...].astype(BF),
                 preferred_element_type=F32).astype(BF)
    cbufr[0] = c0
    cbufl[0] = c0
    kbuf[0] = wuk_ref[...].astype(BF)
    vbuf[0] = wuv_ref[...].astype(BF)

    NS = 2
    CS = BS // NS
    WS = DCS // NS
    paths = ((cbufr, CS, crs, crr, right),
             (kbuf, WS, krs, krr, right),
             (cbufl, CS, cls, clr, left),
             (vbuf, WS, vls, vlr, left))
    rdmas = {}
    for h in range(N_DEV - 1):
        for s in range(NS):
            if h > 0:
                for r in rdmas[(h - 1, s)]:
                    r.wait()
            sub = []
            for buf, rows, ss, rr, tgt in paths:
                r = pltpu.make_async_remote_copy(
                    src_ref=buf.at[h, pl.ds(s * rows, rows)],
                    dst_ref=buf.at[h + 1, pl.ds(s * rows, rows)],
                    send_sem=ss.at[h * NS + s], recv_sem=rr.at[h * NS + s],
                    device_id=(tgt,), device_id_type=pl.DeviceIdType.MESH)
                r.start()
                sub.append(r)
            rdmas[(h, s)] = sub
        if h == 0:
            k_acc = jnp.dot(cbufr[0], kbuf[0], preferred_element_type=F32)
            v_acc = jnp.dot(cbufl[0], vbuf[0], preferred_element_type=F32)
        else:
            k_acc += jnp.dot(cbufr[h], kbuf[h], preferred_element_type=F32)
            v_acc += jnp.dot(cbufl[h], vbuf[h], preferred_element_type=F32)

    for s in range(NS):
        for r in rdmas[(N_DEV - 2, s)]:
            r.wait()
    j = N_DEV - 1
    k_acc += jnp.dot(cbufr[j], kbuf[j], preferred_element_type=F32)
    v_acc += jnp.dot(cbufl[j], vbuf[j], preferred_element_type=F32)
    k_ref[...] = k_acc.astype(BF)
    v_ref[...] = v_acc.astype(BF)


def _proj_body(xb_ref, wq_ref, wqr_ref, wkr_ref, q_ref, qr_ref, kr_ref):
    scale = (DH + DR) ** -0.5
    xb = xb_ref[...]
    q_ref[...] = (jnp.dot(xb, wq_ref[...].astype(BF),
                          preferred_element_type=F32) * scale).astype(BF)
    qr_full = (jnp.dot(xb, wqr_ref[...].astype(BF),
                       preferred_element_type=F32) * scale).astype(BF)
    for hd in range(H):
        qr_ref[hd] = qr_full[:, hd * DR:(hd + 1) * DR]
    kr_ref[...] = jnp.dot(xb, wkr_ref[...].astype(BF),
                          preferred_element_type=F32).astype(BF)


def _attn_body(q_ref, k_ref, v_ref, qr_ref, kr_ref, o_ref):
    q3 = q_ref[...].reshape(B, S, DH)
    k3 = k_ref[...].reshape(B, S, DH)
    qr3 = qr_ref[0].reshape(B, S, DR)
    kr3 = kr_ref[...].reshape(B, S, DR)

    dn = (((2,), (2,)), ((0,), (0,)))
    scores = (lax.dot_general(q3, k3, dn, preferred_element_type=F32)
              + lax.dot_general(qr3, kr3, dn, preferred_element_type=F32))
    p = jnp.exp(scores)
    denom = jnp.sum(p, axis=-1, keepdims=True)

    dn_pv = (((2,), (1,)), ((0,), (0,)))
    o = lax.dot_general(p.astype(BF), v_ref[...].reshape(B, S, DH), dn_pv,
                        preferred_element_type=F32)
    o = o / denom
    o_ref[...] = o.reshape(BS, DH).astype(BF)


def _out_body(o_ref, wo_ref, out_ref):
    j = pl.program_id(0)
    part = jnp.dot(o_ref[...], wo_ref[...].astype(BF),
                   preferred_element_type=F32).reshape(B, S, D)

    @pl.when(j == 0)
    def _():
        out_ref[...] = part

    @pl.when(j != 0)
    def _():
        out_ref[...] += part


def kernel(x, Wdkv, Wuk, Wuv, Wq, Wqr, Wkr, Wo):
    xb, k, v = pl.pallas_call(
        _comm_body,
        out_shape=(
            jax.ShapeDtypeStruct((BS, D), BF),
            jax.ShapeDtypeStruct((BS, D), BF),
            jax.ShapeDtypeStruct((BS, D), BF),
        ),
        in_specs=[pl.BlockSpec(memory_space=pltpu.VMEM)] * 4,
        out_specs=tuple([pl.BlockSpec(memory_space=pltpu.VMEM)] * 3),
        scratch_shapes=[
            pltpu.VMEM((N_DEV, BS, DCS), BF),
            pltpu.VMEM((N_DEV, BS, DCS), BF),
            pltpu.VMEM((N_DEV, DCS, D), BF),
            pltpu.VMEM((N_DEV, DCS, D), BF),
        ] + [pltpu.SemaphoreType.DMA((2 * (N_DEV - 1),))] * 8,
        compiler_params=pltpu.CompilerParams(collective_id=0),
    )(x, Wdkv, Wuk, Wuv)

    q, qr, kr = pl.pallas_call(
        _proj_body,
        in_specs=[pl.BlockSpec(memory_space=pltpu.VMEM)] * 4,
        out_specs=tuple([pl.BlockSpec(memory_space=pltpu.VMEM)] * 3),
        out_shape=(
            jax.ShapeDtypeStruct((BS, D), BF),
            jax.ShapeDtypeStruct((H, BS, DR), BF),
            jax.ShapeDtypeStruct((BS, DR), BF),
        ),
    )(xb, Wq, Wqr, Wkr)

    o = pl.pallas_call(
        _attn_body,
        grid=(H,),
        in_specs=[
            pl.BlockSpec((BS, DH), lambda h: (0, h)),
            pl.BlockSpec((BS, DH), lambda h: (0, h)),
            pl.BlockSpec((BS, DH), lambda h: (0, h)),
            pl.BlockSpec((1, BS, DR), lambda h: (h, 0, 0)),
            pl.BlockSpec((BS, DR), lambda h: (0, 0)),
        ],
        out_specs=pl.BlockSpec((BS, DH), lambda h: (0, h)),
        out_shape=jax.ShapeDtypeStruct((BS, D), BF),
    )(q, k, v, qr, kr)

    out = pl.pallas_call(
        _out_body,
        grid=(2,),
        in_specs=[
            pl.BlockSpec((BS, D // 2), lambda j: (0, j)),
            pl.BlockSpec((D // 2, D), lambda j: (j, 0)),
        ],
        out_specs=pl.BlockSpec((B, S, D), lambda j: (0, 0, 0)),
        out_shape=jax.ShapeDtypeStruct((B, S, D), F32),
    )(o, Wo)
    return out
